# baseline (speedup 1.0000x reference)
import functools

import jax
import jax.numpy as jnp
from jax import lax
from jax.experimental import pallas as pl
from jax.experimental.pallas import tpu as pltpu
from jax.experimental.pallas import tpu_sc as plsc

SLOT_DIM = 142
NUM_ARCHETYPES = 19
HIDDEN_DIM = 256
CODE_DIM = 64
NUM_CODES = 512
COMMIT_COST = 0.25
B = 16384
BB = 4096

_SC_CORES = 2
_SC_SUBCORES = 16
_NW = _SC_CORES * _SC_SUBCORES
_BPW = B // _NW
_CH = 128
_NCH = _BPW // _CH


def _dot(a, b):
    return jax.lax.dot_general(a, b, (((1,), (0,)), ((), ())),
                               preferred_element_type=jnp.float32)


def _enc_body(x_ref, a_ref, w1_ref, b1_ref, w2_ref, b2_ref,
              w3_ref, b3_ref, cbm2_ref, cbn_ref,
              ze_ref, idx_ref):
    x = x_ref[...]
    a = a_ref[...]
    xa = jnp.concatenate([x, a], axis=1)
    h = jnp.maximum(_dot(xa, w1_ref[...]) + b1_ref[...], 0.0)
    h = jnp.maximum(_dot(h, w2_ref[...]) + b2_ref[...], 0.0)
    z_e = _dot(h, w3_ref[...]) + b3_ref[...]
    ze_ref[...] = z_e

    dist = jax.lax.dot_general(z_e, cbm2_ref[...], (((1,), (1,)), ((), ())),
                               preferred_element_type=jnp.float32) + cbn_ref[...]
    dmin = jnp.min(dist, axis=1, keepdims=True)
    iota = jax.lax.broadcasted_iota(jnp.int32, dist.shape, 1)
    idx = jnp.min(jnp.where(dist == dmin, iota, NUM_CODES), axis=1)
    idx_ref[...] = idx[:, None]


def _dec_body(ze_ref, zq_ref, a_ref, wd1_ref, bd1_ref, wd2_ref,
              bd2_ref, wd3_ref, bd3_ref,
              recon_ref, loss_ref):
    z_e = ze_ref[...]
    z_q = zq_ref[...]
    a = a_ref[...]

    diff = z_e - z_q
    loss_ref[...] = jnp.reshape(jnp.sum(diff * diff), (1, 1, 1))

    def _dot_bf16(p, q):
        return jax.lax.dot_general(p.astype(jnp.bfloat16), q.astype(jnp.bfloat16),
                                   (((1,), (0,)), ((), ())),
                                   preferred_element_type=jnp.float32)

    za = jnp.concatenate([z_q, a], axis=1)
    h2 = jnp.maximum(_dot_bf16(za, wd1_ref[...]) + bd1_ref[...], 0.0)
    h2 = jnp.maximum(_dot_bf16(h2, wd2_ref[...]) + bd2_ref[...], 0.0)
    recon_ref[...] = _dot(h2, wd3_ref[...]) + bd3_ref[...]


def _sc_gather(table_hbm, idx_hbm, out_hbm, idx_v, rows_v, sem):
    wid = lax.axis_index("s") * _SC_CORES + lax.axis_index("c")
    base = wid * _BPW
    for j in range(_NCH):
        pltpu.sync_copy(idx_hbm.at[pl.ds(base + j * _CH, _CH)], idx_v.at[j])
    copies = [pltpu.async_copy(table_hbm.at[idx_v.at[j]], rows_v.at[j], sem)
              for j in range(_NCH)]
    for c in copies:
        c.wait()
    for j in range(_NCH):
        pltpu.sync_copy(rows_v.at[j], out_hbm.at[pl.ds(base + j * _CH, _CH)])


def kernel(x, archetype_onehot, W1, b1, W2, b2, W3, b3,
           Wd1, bd1, Wd2, bd2, Wd3, bd3, codebook):
    b1r = b1[None, :]
    b2r = b2[None, :]
    b3r = b3[None, :]
    bd1r = bd1[None, :]
    bd2r = bd2[None, :]
    bd3r = bd3[None, :]
    cbn = jnp.sum(codebook ** 2, axis=1)[None, :]
    cbm2 = -2.0 * codebook

    grid = (B // BB,)
    row = lambda i: (i, 0)
    rep = lambda i: (0, 0)

    def wspec(arr):
        return pl.BlockSpec(arr.shape, rep)

    z_e, idx2d = pl.pallas_call(
        _enc_body,
        grid=grid,
        in_specs=[
            pl.BlockSpec((BB, SLOT_DIM), row),
            pl.BlockSpec((BB, NUM_ARCHETYPES), row),
            wspec(W1), wspec(b1r),
            wspec(W2), wspec(b2r),
            wspec(W3), wspec(b3r),
            wspec(cbm2), wspec(cbn),
        ],
        out_specs=[
            pl.BlockSpec((BB, CODE_DIM), row),
            pl.BlockSpec((BB, 1), row),
        ],
        out_shape=[
            jax.ShapeDtypeStruct((B, CODE_DIM), jnp.float32),
            jax.ShapeDtypeStruct((B, 1), jnp.int32),
        ],
        compiler_params=pltpu.CompilerParams(
            dimension_semantics=("parallel",),
        ),
    )(x, archetype_onehot, W1, b1r, W2, b2r, W3, b3r, cbm2, cbn)

    indices = idx2d[:, 0]

    sc_mesh = plsc.VectorSubcoreMesh(core_axis_name="c", subcore_axis_name="s")
    z_q = pl.kernel(
        _sc_gather,
        mesh=sc_mesh,
        out_type=jax.ShapeDtypeStruct((B, CODE_DIM), jnp.float32),
        scratch_types=[
            pltpu.VMEM((_NCH, _CH), jnp.int32),
            pltpu.VMEM((_NCH, _CH, CODE_DIM), jnp.float32),
            pltpu.SemaphoreType.DMA,
        ],
        compiler_params=pltpu.CompilerParams(use_tc_tiling_on_sc=False),
    )(codebook, indices)

    recon, loss = pl.pallas_call(
        _dec_body,
        grid=grid,
        in_specs=[
            pl.BlockSpec((BB, CODE_DIM), row),
            pl.BlockSpec((BB, CODE_DIM), row),
            pl.BlockSpec((BB, NUM_ARCHETYPES), row),
            wspec(Wd1), wspec(bd1r),
            wspec(Wd2), wspec(bd2r),
            wspec(Wd3), wspec(bd3r),
        ],
        out_specs=[
            pl.BlockSpec((BB, SLOT_DIM), row),
            pl.BlockSpec((1, 1, 1), lambda i: (i, 0, 0)),
        ],
        out_shape=[
            jax.ShapeDtypeStruct((B, SLOT_DIM), jnp.float32),
            jax.ShapeDtypeStruct((B // BB, 1, 1), jnp.float32),
        ],
        compiler_params=pltpu.CompilerParams(
            dimension_semantics=("parallel",),
        ),
    )(z_e, z_q, archetype_onehot, Wd1, bd1r, Wd2, bd2r, Wd3, bd3r)

    vq_loss = (COMMIT_COST / (B * CODE_DIM)) * jnp.sum(loss)
    return (recon, indices, vq_loss)


# speedup vs baseline: 2.4825x; 2.4825x over previous
import jax
import jax.numpy as jnp
from jax.experimental import pallas as pl
from jax.experimental.pallas import tpu as pltpu

SLOT_DIM = 142
NUM_ARCHETYPES = 19
HIDDEN_DIM = 256
CODE_DIM = 64
NUM_CODES = 512
COMMIT_COST = 0.25
B = 16384
BB = 4096


def _dot(a, b):
    return jax.lax.dot_general(a, b, (((1,), (0,)), ((), ())),
                               preferred_element_type=jnp.float32)


def _dot_bf16(p, q):
    return jax.lax.dot_general(p.astype(jnp.bfloat16), q.astype(jnp.bfloat16),
                               (((1,), (0,)), ((), ())),
                               preferred_element_type=jnp.float32)


def _vqvae_body(x_ref, a_ref, w1_ref, b1_ref, w2_ref, b2_ref,
                w3_ref, b3_ref, wd1_ref, bd1_ref, wd2_ref,
                bd2_ref, wd3_ref, bd3_ref, cb_ref, cbm2_ref, cbn_ref,
                recon_ref, idx_ref, loss_ref):
    cb = cb_ref[...]
    cb16 = cb.astype(jnp.bfloat16)

    def _half(sl):
        x = x_ref[sl, :]
        a = a_ref[sl, :]
        xa = jnp.concatenate([x, a], axis=1)
        h = jnp.maximum(_dot(xa, w1_ref[...]) + b1_ref[...], 0.0)
        h = jnp.maximum(_dot(h, w2_ref[...]) + b2_ref[...], 0.0)
        z_e = _dot(h, w3_ref[...]) + b3_ref[...]

        dist = jax.lax.dot_general(z_e, cbm2_ref[...], (((1,), (1,)), ((), ())),
                                   preferred_element_type=jnp.float32) + cbn_ref[...]

        dmin = jnp.min(dist, axis=1, keepdims=True)
        iota = jax.lax.broadcasted_iota(jnp.int32, dist.shape, 1)
        idx = jnp.min(jnp.where(dist == dmin, iota, NUM_CODES), axis=1)
        idx_ref[sl, :] = idx[:, None]

        lpart = jnp.sum(z_e * z_e) + jnp.sum(dmin)

        onehot = (iota == idx[:, None]).astype(jnp.bfloat16)
        z_q = jax.lax.dot_general(onehot, cb16, (((1,), (0,)), ((), ())),
                                  preferred_element_type=jnp.float32)

        za = jnp.concatenate([z_q, a], axis=1)
        h2 = jnp.maximum(_dot_bf16(za, wd1_ref[...]) + bd1_ref[...], 0.0)
        h2 = jnp.maximum(_dot_bf16(h2, wd2_ref[...]) + bd2_ref[...], 0.0)
        recon_ref[sl, :] = _dot(h2, wd3_ref[...]) + bd3_ref[...]
        return lpart

    hb = BB // 2
    l0 = _half(slice(0, hb))
    l1 = _half(slice(hb, BB))
    loss_ref[...] = jnp.reshape(l0 + l1, (1, 1, 1))


def kernel(x, archetype_onehot, W1, b1, W2, b2, W3, b3,
           Wd1, bd1, Wd2, bd2, Wd3, bd3, codebook):
    b1r = b1[None, :]
    b2r = b2[None, :]
    b3r = b3[None, :]
    bd1r = bd1[None, :]
    bd2r = bd2[None, :]
    bd3r = bd3[None, :]
    cbn = jnp.sum(codebook ** 2, axis=1)[None, :]
    cbm2 = -2.0 * codebook

    grid = (B // BB,)
    row = lambda i: (i, 0)
    rep = lambda i: (0, 0)

    def wspec(arr):
        return pl.BlockSpec(arr.shape, rep)

    recon, idx2d, loss = pl.pallas_call(
        _vqvae_body,
        grid=grid,
        in_specs=[
            pl.BlockSpec((BB, SLOT_DIM), row),
            pl.BlockSpec((BB, NUM_ARCHETYPES), row),
            wspec(W1), wspec(b1r),
            wspec(W2), wspec(b2r),
            wspec(W3), wspec(b3r),
            wspec(Wd1), wspec(bd1r),
            wspec(Wd2), wspec(bd2r),
            wspec(Wd3), wspec(bd3r),
            wspec(codebook), wspec(cbm2), wspec(cbn),
        ],
        out_specs=[
            pl.BlockSpec((BB, SLOT_DIM), row),
            pl.BlockSpec((BB, 1), row),
            pl.BlockSpec((1, 1, 1), lambda i: (i, 0, 0)),
        ],
        out_shape=[
            jax.ShapeDtypeStruct((B, SLOT_DIM), jnp.float32),
            jax.ShapeDtypeStruct((B, 1), jnp.int32),
            jax.ShapeDtypeStruct((B // BB, 1, 1), jnp.float32),
        ],
        compiler_params=pltpu.CompilerParams(
            dimension_semantics=("parallel",),
        ),
    )(x, archetype_onehot, W1, b1r, W2, b2r, W3, b3r,
      Wd1, bd1r, Wd2, bd2r, Wd3, bd3r, codebook, cbm2, cbn)

    indices = idx2d[:, 0]
    vq_loss = (COMMIT_COST / (B * CODE_DIM)) * jnp.sum(loss)
    return (recon, indices, vq_loss)


# speedup vs baseline: 2.5372x; 1.0220x over previous
import jax
import jax.numpy as jnp
from jax.experimental import pallas as pl
from jax.experimental.pallas import tpu as pltpu

SLOT_DIM = 142
NUM_ARCHETYPES = 19
HIDDEN_DIM = 256
CODE_DIM = 64
NUM_CODES = 512
COMMIT_COST = 0.25
B = 16384
BB = 4096


def _dot(a, b):
    return jax.lax.dot_general(a, b, (((1,), (0,)), ((), ())),
                               preferred_element_type=jnp.float32)


def _dot_bf16(p, q):
    return jax.lax.dot_general(p.astype(jnp.bfloat16), q.astype(jnp.bfloat16),
                               (((1,), (0,)), ((), ())),
                               preferred_element_type=jnp.float32)


def _vqvae_body(x_ref, a_ref, w1_ref, b1_ref, w2_ref, b2_ref,
                w3_ref, b3_ref, wd1_ref, bd1_ref, wd2_ref,
                bd2_ref, wd3_ref, bd3_ref, cb_ref, cbm2_ref, cbn_ref,
                recon_ref, idx_ref, loss_ref):
    cb = cb_ref[...]
    cb16 = cb.astype(jnp.bfloat16)

    def _half(sl):
        x = x_ref[sl, :]
        a = a_ref[sl, :]
        xa = jnp.concatenate([x, a], axis=1)
        h = jnp.maximum(_dot(xa, w1_ref[...]) + b1_ref[...], 0.0)
        h = jnp.maximum(_dot(h, w2_ref[...]) + b2_ref[...], 0.0)
        z_e = _dot(h, w3_ref[...]) + b3_ref[...]

        dist = jax.lax.dot_general(z_e, cbm2_ref[...], (((1,), (1,)), ((), ())),
                                   preferred_element_type=jnp.float32) + cbn_ref[...]

        dmin = jnp.min(dist, axis=1, keepdims=True)
        iota = jax.lax.broadcasted_iota(jnp.int32, dist.shape, 1)
        idx = jnp.min(jnp.where(dist == dmin, iota, NUM_CODES), axis=1)
        idx_ref[sl, :] = idx[:, None]

        lpart = jnp.sum(z_e * z_e) + jnp.sum(dmin)

        onehot = (iota == idx[:, None]).astype(jnp.bfloat16)
        z_q = jax.lax.dot_general(onehot, cb16, (((1,), (0,)), ((), ())),
                                  preferred_element_type=jnp.float32)

        za = jnp.concatenate([z_q, a], axis=1)
        h2 = jnp.maximum(_dot_bf16(za, wd1_ref[...]) + bd1_ref[...], 0.0)
        h2 = jnp.maximum(_dot_bf16(h2, wd2_ref[...]) + bd2_ref[...], 0.0)
        recon_ref[sl, :] = _dot(h2, wd3_ref[...]) + bd3_ref[...]
        return lpart

    l0 = _half(slice(0, BB))
    loss_ref[...] = jnp.reshape(l0, (1, 1, 1))


def kernel(x, archetype_onehot, W1, b1, W2, b2, W3, b3,
           Wd1, bd1, Wd2, bd2, Wd3, bd3, codebook):
    b1r = b1[None, :]
    b2r = b2[None, :]
    b3r = b3[None, :]
    bd1r = bd1[None, :]
    bd2r = bd2[None, :]
    bd3r = bd3[None, :]
    cbn = jnp.sum(codebook ** 2, axis=1)[None, :]
    cbm2 = -2.0 * codebook

    grid = (B // BB,)
    row = lambda i: (i, 0)
    rep = lambda i: (0, 0)

    def wspec(arr):
        return pl.BlockSpec(arr.shape, rep)

    recon, idx2d, loss = pl.pallas_call(
        _vqvae_body,
        grid=grid,
        in_specs=[
            pl.BlockSpec((BB, SLOT_DIM), row),
            pl.BlockSpec((BB, NUM_ARCHETYPES), row),
            wspec(W1), wspec(b1r),
            wspec(W2), wspec(b2r),
            wspec(W3), wspec(b3r),
            wspec(Wd1), wspec(bd1r),
            wspec(Wd2), wspec(bd2r),
            wspec(Wd3), wspec(bd3r),
            wspec(codebook), wspec(cbm2), wspec(cbn),
        ],
        out_specs=[
            pl.BlockSpec((BB, SLOT_DIM), row),
            pl.BlockSpec((BB, 1), row),
            pl.BlockSpec((1, 1, 1), lambda i: (i, 0, 0)),
        ],
        out_shape=[
            jax.ShapeDtypeStruct((B, SLOT_DIM), jnp.float32),
            jax.ShapeDtypeStruct((B, 1), jnp.int32),
            jax.ShapeDtypeStruct((B // BB, 1, 1), jnp.float32),
        ],
        compiler_params=pltpu.CompilerParams(
            dimension_semantics=("parallel",),
        ),
    )(x, archetype_onehot, W1, b1r, W2, b2r, W3, b3r,
      Wd1, bd1r, Wd2, bd2r, Wd3, bd3r, codebook, cbm2, cbn)

    indices = idx2d[:, 0]
    vq_loss = (COMMIT_COST / (B * CODE_DIM)) * jnp.sum(loss)
    return (recon, indices, vq_loss)


# speedup vs baseline: 2.5680x; 1.0122x over previous
import jax
import jax.numpy as jnp
from jax.experimental import pallas as pl
from jax.experimental.pallas import tpu as pltpu

SLOT_DIM = 142
NUM_ARCHETYPES = 19
HIDDEN_DIM = 256
CODE_DIM = 64
NUM_CODES = 512
COMMIT_COST = 0.25
B = 16384
BB = 4096


def _dot(a, b):
    return jax.lax.dot_general(a, b, (((1,), (0,)), ((), ())),
                               preferred_element_type=jnp.float32)


def _dot_bf16(p, q):
    return jax.lax.dot_general(p.astype(jnp.bfloat16), q.astype(jnp.bfloat16),
                               (((1,), (0,)), ((), ())),
                               preferred_element_type=jnp.float32)


def _vqvae_body(x_ref, a_ref, w1_ref, b1_ref, w2_ref, b2_ref,
                w3_ref, b3_ref, wd1_ref, bd1_ref, wd2_ref,
                bd2_ref, wd3_ref, bd3_ref, cb_ref, cbm2_ref, cbn_ref,
                iota_ref, recon_ref, idx_ref, loss_ref):
    cb = cb_ref[...]
    cb16 = cb.astype(jnp.bfloat16)

    def _half(sl):
        x = x_ref[sl, :]
        a = a_ref[sl, :]
        xa = jnp.concatenate([x, a], axis=1)
        h = jnp.maximum(_dot(xa, w1_ref[...]) + b1_ref[...], 0.0)
        h = jnp.maximum(_dot(h, w2_ref[...]) + b2_ref[...], 0.0)
        z_e = _dot(h, w3_ref[...]) + b3_ref[...]

        dist = jax.lax.dot_general(z_e, cbm2_ref[...], (((1,), (1,)), ((), ())),
                                   preferred_element_type=jnp.float32) + cbn_ref[...]

        dmin = jnp.min(dist, axis=1, keepdims=True)
        iota = iota_ref[...]
        idxf = jnp.min(jnp.where(dist == dmin, iota, float(NUM_CODES)), axis=1)
        idx_ref[sl, :] = idxf.astype(jnp.int32)[:, None]

        lpart = jnp.sum(z_e * z_e) + jnp.sum(dmin)

        onehot = (iota == idxf[:, None]).astype(jnp.bfloat16)
        z_q = jax.lax.dot_general(onehot, cb16, (((1,), (0,)), ((), ())),
                                  preferred_element_type=jnp.float32)

        za = jnp.concatenate([z_q, a], axis=1)
        h2 = jnp.maximum(_dot_bf16(za, wd1_ref[...]) + bd1_ref[...], 0.0)
        h2 = jnp.maximum(_dot_bf16(h2, wd2_ref[...]) + bd2_ref[...], 0.0)
        recon_ref[sl, :] = _dot(h2, wd3_ref[...]) + bd3_ref[...]
        return lpart

    l0 = _half(slice(0, BB))
    loss_ref[...] = jnp.reshape(l0, (1, 1, 1))


def kernel(x, archetype_onehot, W1, b1, W2, b2, W3, b3,
           Wd1, bd1, Wd2, bd2, Wd3, bd3, codebook):
    b1r = b1[None, :]
    b2r = b2[None, :]
    b3r = b3[None, :]
    bd1r = bd1[None, :]
    bd2r = bd2[None, :]
    bd3r = bd3[None, :]
    cbn = jnp.sum(codebook ** 2, axis=1)[None, :]
    cbm2 = -2.0 * codebook
    iota_row = jnp.arange(NUM_CODES, dtype=jnp.float32)[None, :]

    grid = (B // BB,)
    row = lambda i: (i, 0)
    rep = lambda i: (0, 0)

    def wspec(arr):
        return pl.BlockSpec(arr.shape, rep)

    recon, idx2d, loss = pl.pallas_call(
        _vqvae_body,
        grid=grid,
        in_specs=[
            pl.BlockSpec((BB, SLOT_DIM), row),
            pl.BlockSpec((BB, NUM_ARCHETYPES), row),
            wspec(W1), wspec(b1r),
            wspec(W2), wspec(b2r),
            wspec(W3), wspec(b3r),
            wspec(Wd1), wspec(bd1r),
            wspec(Wd2), wspec(bd2r),
            wspec(Wd3), wspec(bd3r),
            wspec(codebook), wspec(cbm2), wspec(cbn), wspec(iota_row),
        ],
        out_specs=[
            pl.BlockSpec((BB, SLOT_DIM), row),
            pl.BlockSpec((BB, 1), row),
            pl.BlockSpec((1, 1, 1), lambda i: (i, 0, 0)),
        ],
        out_shape=[
            jax.ShapeDtypeStruct((B, SLOT_DIM), jnp.float32),
            jax.ShapeDtypeStruct((B, 1), jnp.int32),
            jax.ShapeDtypeStruct((B // BB, 1, 1), jnp.float32),
        ],
        compiler_params=pltpu.CompilerParams(
            dimension_semantics=("parallel",),
        ),
    )(x, archetype_onehot, W1, b1r, W2, b2r, W3, b3r,
      Wd1, bd1r, Wd2, bd2r, Wd3, bd3r, codebook, cbm2, cbn, iota_row)

    indices = idx2d[:, 0]
    vq_loss = (COMMIT_COST / (B * CODE_DIM)) * jnp.sum(loss)
    return (recon, indices, vq_loss)


# speedup vs baseline: 2.6210x; 1.0206x over previous
import jax
import jax.numpy as jnp
from jax.experimental import pallas as pl
from jax.experimental.pallas import tpu as pltpu

SLOT_DIM = 142
NUM_ARCHETYPES = 19
HIDDEN_DIM = 256
CODE_DIM = 64
NUM_CODES = 512
COMMIT_COST = 0.25
B = 16384
BB = 4096


def _dot(a, b):
    return jax.lax.dot_general(a, b, (((1,), (0,)), ((), ())),
                               preferred_element_type=jnp.float32)


def _dot_bf16(p, q):
    return jax.lax.dot_general(p.astype(jnp.bfloat16), q.astype(jnp.bfloat16),
                               (((1,), (0,)), ((), ())),
                               preferred_element_type=jnp.float32)


def _vqvae_body(x_ref, a_ref, w1_ref, b1_ref, w2_ref, b2_ref,
                w3_ref, b3_ref, wd1_ref, bd1_ref, wd2_ref,
                bd2_ref, wd3_ref, bd3_ref, cb_ref, cbm2_ref, cbn_ref,
                iota_ref, recon_ref, idx_ref, loss_ref):
    cb = cb_ref[...]

    def _half(sl):
        x = x_ref[sl, :]
        a = a_ref[sl, :]
        xa = jnp.concatenate([x, a], axis=1)
        h = jnp.maximum(_dot(xa, w1_ref[...]) + b1_ref[...], 0.0)
        h = jnp.maximum(_dot(h, w2_ref[...]) + b2_ref[...], 0.0)
        z_e = _dot(h, w3_ref[...]) + b3_ref[...]

        dist = jax.lax.dot_general(z_e, cbm2_ref[...], (((1,), (1,)), ((), ())),
                                   preferred_element_type=jnp.float32) + cbn_ref[...]

        dmin = jnp.min(dist, axis=1, keepdims=True)
        iota = iota_ref[...]
        idxf = jnp.min(jnp.where(dist == dmin, iota, float(NUM_CODES)), axis=1)
        idx_ref[sl, :] = idxf.astype(jnp.int32)[:, None]

        lpart = jnp.sum(z_e * z_e) + jnp.sum(dmin)

        onehot = (iota == idxf[:, None]).astype(jnp.float32)
        z_q = _dot(onehot, cb)

        za = jnp.concatenate([z_q, a], axis=1)
        h2 = jnp.maximum(_dot(za, wd1_ref[...]) + bd1_ref[...], 0.0)
        h2 = jnp.maximum(_dot(h2, wd2_ref[...]) + bd2_ref[...], 0.0)
        recon_ref[sl, :] = _dot(h2, wd3_ref[...]) + bd3_ref[...]
        return lpart

    l0 = _half(slice(0, BB))
    loss_ref[...] = jnp.reshape(l0, (1, 1, 1))


def kernel(x, archetype_onehot, W1, b1, W2, b2, W3, b3,
           Wd1, bd1, Wd2, bd2, Wd3, bd3, codebook):
    b1r = b1[None, :]
    b2r = b2[None, :]
    b3r = b3[None, :]
    bd1r = bd1[None, :]
    bd2r = bd2[None, :]
    bd3r = bd3[None, :]
    cbn = jnp.sum(codebook ** 2, axis=1)[None, :]
    cbm2 = -2.0 * codebook
    iota_row = jnp.arange(NUM_CODES, dtype=jnp.float32)[None, :]

    grid = (B // BB,)
    row = lambda i: (i, 0)
    rep = lambda i: (0, 0)

    def wspec(arr):
        return pl.BlockSpec(arr.shape, rep)

    recon, idx2d, loss = pl.pallas_call(
        _vqvae_body,
        grid=grid,
        in_specs=[
            pl.BlockSpec((BB, SLOT_DIM), row),
            pl.BlockSpec((BB, NUM_ARCHETYPES), row),
            wspec(W1), wspec(b1r),
            wspec(W2), wspec(b2r),
            wspec(W3), wspec(b3r),
            wspec(Wd1), wspec(bd1r),
            wspec(Wd2), wspec(bd2r),
            wspec(Wd3), wspec(bd3r),
            wspec(codebook), wspec(cbm2), wspec(cbn), wspec(iota_row),
        ],
        out_specs=[
            pl.BlockSpec((BB, SLOT_DIM), row),
            pl.BlockSpec((BB, 1), row),
            pl.BlockSpec((1, 1, 1), lambda i: (i, 0, 0)),
        ],
        out_shape=[
            jax.ShapeDtypeStruct((B, SLOT_DIM), jnp.float32),
            jax.ShapeDtypeStruct((B, 1), jnp.int32),
            jax.ShapeDtypeStruct((B // BB, 1, 1), jnp.float32),
        ],
        compiler_params=pltpu.CompilerParams(
            dimension_semantics=("parallel",),
        ),
    )(x, archetype_onehot, W1, b1r, W2, b2r, W3, b3r,
      Wd1, bd1r, Wd2, bd2r, Wd3, bd3r, codebook, cbm2, cbn, iota_row)

    indices = idx2d[:, 0]
    vq_loss = (COMMIT_COST / (B * CODE_DIM)) * jnp.sum(loss)
    return (recon, indices, vq_loss)
